# baseline (speedup 1.0000x reference)
import functools
import math

import jax
import jax.numpy as jnp
from jax import lax
from jax.experimental import pallas as pl
from jax.experimental.pallas import tpu as pltpu
from jax.experimental.pallas import tpu_sc as plsc

VOCAB = 1000000
EMBED = 64
LANES = 16
NUM_CORES = 2
NUM_SUBCORES = 16
NUM_WORKERS = NUM_CORES * NUM_SUBCORES
IDX_BLK = 128
SCALE = math.sqrt(EMBED)


def _emb_lookup(table2, idx3, off3, b, l):
    nw, n_blocks, blk = idx3.shape
    g_per_l = b // (NUM_WORKERS * IDX_BLK)

    mesh = plsc.VectorSubcoreMesh(core_axis_name="c", subcore_axis_name="s")

    @functools.partial(
        pl.kernel,
        mesh=mesh,
        out_type=jax.ShapeDtypeStruct((l, EMBED, b), jnp.float32),
        scratch_types=[
            pltpu.VMEM((n_blocks, blk), jnp.int32),
            pltpu.VMEM((n_blocks, blk), jnp.int32),
            pltpu.VMEM((blk, 2 * EMBED), jnp.float32),
            pltpu.VMEM((blk, 2 * EMBED), jnp.float32),
            pltpu.VMEM((EMBED, blk), jnp.float32),
            pltpu.VMEM((EMBED, blk), jnp.float32),
            pltpu.SemaphoreType.DMA,
            pltpu.SemaphoreType.DMA,
            pltpu.SemaphoreType.DMA,
            pltpu.SemaphoreType.DMA,
        ],
        compiler_params=pltpu.CompilerParams(needs_layout_passes=False),
    )
    def k(tab_hbm, idx_hbm, off_hbm, out_hbm, idx_v, off_v,
          gbuf0, gbuf1, obuf0, obuf1, gs0, gs1, ws0, ws1):
        gbufs = (gbuf0, gbuf1)
        obufs = (obuf0, obuf1)
        gsems = (gs0, gs1)
        wsems = (ws0, ws1)
        wid = lax.axis_index("s") * NUM_CORES + lax.axis_index("c")
        base_b = wid * (g_per_l * IDX_BLK)
        pltpu.sync_copy(idx_hbm.at[wid], idx_v)
        pltpu.sync_copy(off_hbm.at[wid], off_v)

        def out_slice(j):
            pos = j // g_per_l
            sub = j % g_per_l
            col = base_b + sub * IDX_BLK
            return out_hbm.at[pos, :, pl.ds(col, IDX_BLK)]

        def issue_gather(j, bb):
            pltpu.async_copy(tab_hbm.at[idx_v.at[j]], gbufs[bb], gsems[bb])

        def wait_gather(j, bb):
            pltpu.make_async_copy(
                tab_hbm.at[idx_v.at[j]], gbufs[bb], gsems[bb]
            ).wait()

        def issue_write(j, bb):
            pltpu.async_copy(obufs[bb], out_slice(j), wsems[bb])

        def wait_write(j, bb):
            pltpu.make_async_copy(obufs[bb], out_slice(j), wsems[bb]).wait()

        iota16 = lax.iota(jnp.int32, 16)

        def process(j, bb):
            gbuf = gbufs[bb]
            obuf = obufs[bb]
            for kk in range(IDX_BLK // LANES):
                rowv = iota16 + (kk * LANES)
                pv = off_v[j, pl.ds(kk * LANES, LANES)]

                def e_body(e, carry, _kk=kk, _rowv=rowv, _pv=pv,
                           _gbuf=gbuf, _obuf=obuf):
                    colv = _pv + e
                    g16 = plsc.load_gather(_gbuf, [_rowv, colv])
                    _obuf[e, pl.ds(_kk * LANES, LANES)] = g16 * SCALE
                    return carry

                lax.fori_loop(0, EMBED, e_body, 0, unroll=4)

        for bb in range(2):
            issue_gather(bb, bb)

        for bb in range(2):
            wait_gather(bb, bb)
            process(bb, bb)
            issue_write(bb, bb)
            issue_gather(bb + 2, bb)

        def body(i, carry):
            for bb in range(2):
                j = 2 * i + bb
                wait_gather(j, bb)
                wait_write(j - 2, bb)
                process(j, bb)
                issue_write(j, bb)
                issue_gather(j + 2, bb)
            return carry

        lax.fori_loop(1, n_blocks // 2 - 1, body, 0)

        for bb in range(2):
            j = n_blocks - 2 + bb
            wait_gather(j, bb)
            wait_write(j - 2, bb)
            process(j, bb)
            issue_write(j, bb)

        for bb in range(2):
            wait_write(n_blocks - 2 + bb, bb)

    return k(table2, idx3, off3)


def kernel(input, table):
    b, l = input.shape
    idx = input.astype(jnp.int32)
    table2 = table.reshape(VOCAB // 2, 2 * EMBED)
    idxT = idx.T
    g_per_l = b // (NUM_WORKERS * IDX_BLK)
    n_blocks = l * g_per_l

    def arrange(a):
        a4 = a.reshape(l, NUM_WORKERS, g_per_l, IDX_BLK)
        return a4.transpose(1, 0, 2, 3).reshape(NUM_WORKERS, n_blocks, IDX_BLK)

    idx3 = arrange(idxT >> 1)
    off3 = arrange((idxT & 1) * EMBED)
    out_phys = _emb_lookup(table2, idx3, off3, b, l)
    return jnp.transpose(out_phys, (2, 0, 1))


# speedup vs baseline: 1.5267x; 1.5267x over previous
import functools
import math

import jax
import jax.numpy as jnp
from jax import lax
from jax.experimental import pallas as pl
from jax.experimental.pallas import tpu as pltpu
from jax.experimental.pallas import tpu_sc as plsc

VOCAB = 1000000
EMBED = 64
LANES = 16
NUM_CORES = 2
NUM_SUBCORES = 16
NUM_WORKERS = NUM_CORES * NUM_SUBCORES
IDX_BLK = 128
SCALE = math.sqrt(EMBED)


def _emb_lookup(table2, idx3, off3, b, l):
    nw, n_blocks, blk = idx3.shape
    g_per_l = b // (NUM_WORKERS * IDX_BLK)

    mesh = plsc.VectorSubcoreMesh(core_axis_name="c", subcore_axis_name="s")

    @functools.partial(
        pl.kernel,
        mesh=mesh,
        out_type=jax.ShapeDtypeStruct((l, EMBED, b), jnp.float32),
        scratch_types=[
            pltpu.VMEM((n_blocks, blk), jnp.int32),
            pltpu.VMEM((n_blocks, blk), jnp.int32),
            pltpu.VMEM((blk, 2 * EMBED), jnp.float32),
            pltpu.VMEM((blk, 2 * EMBED), jnp.float32),
            pltpu.VMEM((EMBED, blk), jnp.float32),
            pltpu.VMEM((EMBED, blk), jnp.float32),
            pltpu.SemaphoreType.DMA,
            pltpu.SemaphoreType.DMA,
            pltpu.SemaphoreType.DMA,
            pltpu.SemaphoreType.DMA,
        ],
        compiler_params=pltpu.CompilerParams(needs_layout_passes=False),
    )
    def k(tab_hbm, idx_hbm, off_hbm, out_hbm, idx_v, off_v,
          gbuf0, gbuf1, obuf0, obuf1, gs0, gs1, ws0, ws1):
        gbufs = (gbuf0, gbuf1)
        obufs = (obuf0, obuf1)
        gsems = (gs0, gs1)
        wsems = (ws0, ws1)
        wid = lax.axis_index("s") * NUM_CORES + lax.axis_index("c")
        base_b = wid * (g_per_l * IDX_BLK)
        pltpu.sync_copy(idx_hbm.at[wid], idx_v)
        pltpu.sync_copy(off_hbm.at[wid], off_v)

        def out_slice(j):
            pos = j // g_per_l
            sub = j % g_per_l
            col = base_b + sub * IDX_BLK
            return out_hbm.at[pos, :, pl.ds(col, IDX_BLK)]

        def issue_gather(j, bb):
            pltpu.async_copy(tab_hbm.at[idx_v.at[j]], gbufs[bb], gsems[bb])

        def wait_gather(j, bb):
            pltpu.make_async_copy(
                tab_hbm.at[idx_v.at[j]], gbufs[bb], gsems[bb]
            ).wait()

        def issue_write(j, bb):
            pltpu.async_copy(obufs[bb], out_slice(j), wsems[bb])

        def wait_write(j, bb):
            pltpu.make_async_copy(obufs[bb], out_slice(j), wsems[bb]).wait()

        iota16 = lax.iota(jnp.int32, 16)

        def process(j, bb):
            gbuf = gbufs[bb]
            obuf = obufs[bb]
            for kk in range(IDX_BLK // LANES):
                rowv = iota16 + (kk * LANES)
                pv = off_v[j, pl.ds(kk * LANES, LANES)]

                @plsc.parallel_loop(0, EMBED, unroll=8)
                def e_body(e, _kk=kk, _rowv=rowv, _pv=pv,
                           _gbuf=gbuf, _obuf=obuf):
                    colv = _pv + e
                    g16 = plsc.load_gather(_gbuf, [_rowv, colv])
                    _obuf[e, pl.ds(_kk * LANES, LANES)] = g16 * SCALE

        for bb in range(2):
            issue_gather(bb, bb)

        for bb in range(2):
            wait_gather(bb, bb)
            process(bb, bb)
            issue_write(bb, bb)
            issue_gather(bb + 2, bb)

        def body(i, carry):
            for bb in range(2):
                j = 2 * i + bb
                wait_gather(j, bb)
                wait_write(j - 2, bb)
                process(j, bb)
                issue_write(j, bb)
                issue_gather(j + 2, bb)
            return carry

        lax.fori_loop(1, n_blocks // 2 - 1, body, 0)

        for bb in range(2):
            j = n_blocks - 2 + bb
            wait_gather(j, bb)
            wait_write(j - 2, bb)
            process(j, bb)
            issue_write(j, bb)

        for bb in range(2):
            wait_write(n_blocks - 2 + bb, bb)

    return k(table2, idx3, off3)


def kernel(input, table):
    b, l = input.shape
    idx = input.astype(jnp.int32)
    table2 = table.reshape(VOCAB // 2, 2 * EMBED)
    idxT = idx.T
    g_per_l = b // (NUM_WORKERS * IDX_BLK)
    n_blocks = l * g_per_l

    def arrange(a):
        a4 = a.reshape(l, NUM_WORKERS, g_per_l, IDX_BLK)
        return a4.transpose(1, 0, 2, 3).reshape(NUM_WORKERS, n_blocks, IDX_BLK)

    idx3 = arrange(idxT >> 1)
    off3 = arrange((idxT & 1) * EMBED)
    out_phys = _emb_lookup(table2, idx3, off3, b, l)
    return jnp.transpose(out_phys, (2, 0, 1))


# speedup vs baseline: 1.5306x; 1.0025x over previous
import functools
import math

import jax
import jax.numpy as jnp
from jax import lax
from jax.experimental import pallas as pl
from jax.experimental.pallas import tpu as pltpu
from jax.experimental.pallas import tpu_sc as plsc

VOCAB = 1000000
EMBED = 64
LANES = 16
NUM_CORES = 2
NUM_SUBCORES = 16
NUM_WORKERS = NUM_CORES * NUM_SUBCORES
IDX_BLK = 128
SCALE = math.sqrt(EMBED)


def _emb_lookup(table2, idx3, off3, b, l):
    nw, n_blocks, blk = idx3.shape
    g_per_l = b // (NUM_WORKERS * IDX_BLK)

    mesh = plsc.VectorSubcoreMesh(core_axis_name="c", subcore_axis_name="s")

    @functools.partial(
        pl.kernel,
        mesh=mesh,
        out_type=jax.ShapeDtypeStruct((l, EMBED, b), jnp.float32),
        scratch_types=[
            pltpu.VMEM((n_blocks, blk), jnp.int32),
            pltpu.VMEM((n_blocks, blk), jnp.int32),
            pltpu.VMEM((blk, 2 * EMBED), jnp.float32),
            pltpu.VMEM((blk, 2 * EMBED), jnp.float32),
            pltpu.VMEM((EMBED, blk), jnp.float32),
            pltpu.VMEM((EMBED, blk), jnp.float32),
            pltpu.SemaphoreType.DMA,
            pltpu.SemaphoreType.DMA,
            pltpu.SemaphoreType.DMA,
            pltpu.SemaphoreType.DMA,
        ],
        compiler_params=pltpu.CompilerParams(needs_layout_passes=False),
    )
    def k(tab_hbm, idx_hbm, off_hbm, out_hbm, idx_v, off_v,
          gbuf0, gbuf1, obuf0, obuf1, gs0, gs1, ws0, ws1):
        gbufs = (gbuf0, gbuf1)
        obufs = (obuf0, obuf1)
        gsems = (gs0, gs1)
        wsems = (ws0, ws1)
        wid = lax.axis_index("s") * NUM_CORES + lax.axis_index("c")
        base_b = wid * (g_per_l * IDX_BLK)
        pltpu.sync_copy(idx_hbm.at[wid], idx_v)
        pltpu.sync_copy(off_hbm.at[wid], off_v)

        def out_slice(j):
            pos = j // g_per_l
            sub = j % g_per_l
            col = base_b + sub * IDX_BLK
            return out_hbm.at[pos, :, pl.ds(col, IDX_BLK)]

        def issue_gather(j, bb):
            pltpu.async_copy(tab_hbm.at[idx_v.at[j]], gbufs[bb], gsems[bb])

        def wait_gather(j, bb):
            pltpu.make_async_copy(
                tab_hbm.at[idx_v.at[j]], gbufs[bb], gsems[bb]
            ).wait()

        def issue_write(j, bb):
            pltpu.async_copy(obufs[bb], out_slice(j), wsems[bb])

        def wait_write(j, bb):
            pltpu.make_async_copy(obufs[bb], out_slice(j), wsems[bb]).wait()

        iota16 = lax.iota(jnp.int32, 16)

        def process(j, bb):
            gbuf = gbufs[bb]
            obuf = obufs[bb]
            for kk in range(IDX_BLK // LANES):
                rowv = iota16 + (kk * LANES)
                pv = off_v[j, pl.ds(kk * LANES, LANES)]

                @plsc.parallel_loop(0, EMBED, unroll=16)
                def e_body(e, _kk=kk, _rowv=rowv, _pv=pv,
                           _gbuf=gbuf, _obuf=obuf):
                    colv = _pv + e
                    g16 = plsc.load_gather(_gbuf, [_rowv, colv])
                    _obuf[e, pl.ds(_kk * LANES, LANES)] = g16 * SCALE

        for bb in range(2):
            issue_gather(bb, bb)

        for bb in range(2):
            wait_gather(bb, bb)
            process(bb, bb)
            issue_write(bb, bb)
            issue_gather(bb + 2, bb)

        def body(i, carry):
            for bb in range(2):
                j = 2 * i + bb
                wait_gather(j, bb)
                wait_write(j - 2, bb)
                process(j, bb)
                issue_write(j, bb)
                issue_gather(j + 2, bb)
            return carry

        lax.fori_loop(1, n_blocks // 2 - 1, body, 0)

        for bb in range(2):
            j = n_blocks - 2 + bb
            wait_gather(j, bb)
            wait_write(j - 2, bb)
            process(j, bb)
            issue_write(j, bb)

        for bb in range(2):
            wait_write(n_blocks - 2 + bb, bb)

    return k(table2, idx3, off3)


def kernel(input, table):
    b, l = input.shape
    idx = input.astype(jnp.int32)
    table2 = table.reshape(VOCAB // 2, 2 * EMBED)
    idxT = idx.T
    g_per_l = b // (NUM_WORKERS * IDX_BLK)
    n_blocks = l * g_per_l

    def arrange(a):
        a4 = a.reshape(l, NUM_WORKERS, g_per_l, IDX_BLK)
        return a4.transpose(1, 0, 2, 3).reshape(NUM_WORKERS, n_blocks, IDX_BLK)

    idx3 = arrange(idxT >> 1)
    off3 = arrange((idxT & 1) * EMBED)
    out_phys = _emb_lookup(table2, idx3, off3, b, l)
    return jnp.transpose(out_phys, (2, 0, 1))
